# baseline (speedup 1.0000x reference)
import jax
import jax.numpy as jnp
from jax import lax
from jax.experimental import pallas as pl
from jax.experimental.pallas import tpu as pltpu
from jax.experimental.pallas import tpu_sc as plsc

N_UNIQUE = 1000000
DIM_EMB = 16
BATCH = 16384
N_FIELDS = 26

R = BATCH * N_FIELDS
NW = 32
RW = R // NW
CHUNK = 128
K = 13
SUPER = K * CHUNK
NSUPER = RW // SUPER

NTCOL = (N_UNIQUE + 127) // 128
NFULL = N_UNIQUE // 128
TAIL = N_UNIQUE - NFULL * 128
COLS_PER_W = (NTCOL + NW - 1) // NW


def _transpose_body(wt_hbm, tail_hbm, lin_hbm, in_v, out_v, sem):
    nc = 2
    wid = lax.axis_index("s") * nc + lax.axis_index("c")
    lanes = lax.iota(jnp.int32, 16)

    def xpose(n, _):
        def col(c, _):
            row = plsc.load_gather(in_v, [lanes, jnp.zeros((16,), jnp.int32) + c])
            out_v[pl.ds(c * 16, 16)] = row
            return 0

        lax.fori_loop(0, n, col, 0)

    def step(i, _):
        j = i * NW + wid

        @pl.when(j < NFULL)
        def _():
            off = pl.multiple_of(j * 128, 128)
            pltpu.sync_copy(wt_hbm.at[:, pl.ds(off, 128)], in_v)
            xpose(128, None)
            pltpu.sync_copy(out_v, lin_hbm.at[pl.ds(off * 16, 2048)])

        @pl.when(j == NFULL)
        def _():
            pltpu.sync_copy(tail_hbm, out_v.at[pl.ds(0, TAIL * DIM_EMB)])
            pltpu.sync_copy(
                out_v.at[pl.ds(0, TAIL * DIM_EMB)],
                lin_hbm.at[pl.ds(NFULL * 128 * DIM_EMB, TAIL * DIM_EMB)],
            )

        return 0

    lax.fori_loop(0, COLS_PER_W, step, 0)


def _gather_body(table_hbm, ids_hbm, out_hbm, idx_v, rows_v, sem0, sem1):
    nc = 2
    wid = lax.axis_index("s") * nc + lax.axis_index("c")
    base = wid * RW
    pltpu.sync_copy(ids_hbm.at[pl.ds(base, RW)], idx_v)

    def fire(s, b, sem):
        soff = pl.multiple_of(s * SUPER, 8)
        for c in range(K):
            pltpu.async_copy(
                table_hbm.at[idx_v.at[pl.ds(soff + c * CHUNK, CHUNK)]],
                rows_v.at[b, pl.ds(c * CHUNK, CHUNK)],
                sem,
            )

    def drain(b, sem):
        for c in range(K):
            pltpu.make_async_copy(
                table_hbm.at[idx_v.at[pl.ds(c * CHUNK, CHUNK)]],
                rows_v.at[b, pl.ds(c * CHUNK, CHUNK)],
                sem,
            ).wait()

    def flush(s, b):
        ooff = pl.multiple_of(base + s * SUPER, 8)
        pltpu.sync_copy(rows_v.at[b], out_hbm.at[pl.ds(ooff, SUPER)])

    fire(0, 0, sem0)

    def pair(p, _):
        s0 = p * 2
        fire(s0 + 1, 1, sem1)
        drain(0, sem0)
        flush(s0, 0)

        @pl.when(s0 + 2 < NSUPER)
        def _():
            fire(s0 + 2, 0, sem0)

        drain(1, sem1)
        flush(s0 + 1, 1)
        return 0

    lax.fori_loop(0, NSUPER // 2, pair, 0)


@jax.jit
def _encode(emb_weight, flat_ids):
    mesh = plsc.VectorSubcoreMesh(core_axis_name="c", subcore_axis_name="s")
    detile = pl.kernel(
        _transpose_body,
        out_type=jax.ShapeDtypeStruct((N_UNIQUE * DIM_EMB,), jnp.float32),
        mesh=mesh,
        scratch_types=[
            pltpu.VMEM((DIM_EMB, 128), jnp.float32),
            pltpu.VMEM((128 * DIM_EMB,), jnp.float32),
            pltpu.SemaphoreType.DMA,
        ],
        compiler_params=pltpu.CompilerParams(needs_layout_passes=False),
    )
    tail = emb_weight[NFULL * 128 :].reshape(-1)
    lin = detile(emb_weight.T, tail)
    table = lin.reshape(N_UNIQUE, DIM_EMB)

    gather = pl.kernel(
        _gather_body,
        out_type=jax.ShapeDtypeStruct((R, DIM_EMB), jnp.float32),
        mesh=mesh,
        scratch_types=[
            pltpu.VMEM((RW,), jnp.int32),
            pltpu.VMEM((2, SUPER, DIM_EMB), jnp.float32),
            pltpu.SemaphoreType.DMA,
            pltpu.SemaphoreType.DMA,
        ],
        compiler_params=pltpu.CompilerParams(use_tc_tiling_on_sc=False),
    )
    return gather(table, flat_ids)


def kernel(state_ids, emb_weight):
    flat_ids = state_ids.reshape(-1)
    out = _encode(emb_weight, flat_ids)
    return out.reshape(BATCH, N_FIELDS * DIM_EMB)


# speedup vs baseline: 1.3904x; 1.3904x over previous
import jax
import jax.numpy as jnp
from jax import lax
from jax.experimental import pallas as pl
from jax.experimental.pallas import tpu as pltpu
from jax.experimental.pallas import tpu_sc as plsc

N_UNIQUE = 1000000
DIM_EMB = 16
BATCH = 16384
N_FIELDS = 26

R = BATCH * N_FIELDS
NW = 32
RW = R // NW
CHUNK = 128
K = 13
SUPER = K * CHUNK
NSUPER = RW // SUPER

NTCOL = (N_UNIQUE + 127) // 128
NFULL = N_UNIQUE // 128
TAIL = N_UNIQUE - NFULL * 128
COLS_PER_W = (NTCOL + NW - 1) // NW


def _transpose_body(
    wt_hbm, tail_hbm, lin_hbm, in0, in1, out0, out1, si0, si1, so0, so1
):
    nc = 2
    wid = lax.axis_index("s") * nc + lax.axis_index("c")
    lanes = lax.iota(jnp.int32, 16)
    lanes128 = lanes * 128
    ins = (in0, in1)
    outs = (out0, out1)
    sis = (si0, si1)
    sos = (so0, so1)

    def fire_in(i, q):
        off = pl.multiple_of((i * NW + wid) * 128, 128)
        pltpu.async_copy(wt_hbm.at[:, pl.ds(off, 128)], ins[q], sis[q])

    fire_in(0, 0)

    def body(p, _):
        for q in (0, 1):
            i = p * 2 + q
            j = i * NW + wid

            @pl.when(j < NFULL)
            def _():
                @pl.when(j + NW < NFULL)
                def _():
                    fire_in(i + 1, 1 - q)

                pltpu.make_async_copy(
                    wt_hbm.at[:, pl.ds(0, 128)], ins[q], sis[q]
                ).wait()

                @pl.when(j >= 2 * NW)
                def _():
                    pltpu.make_async_copy(
                        outs[q], lin_hbm.at[pl.ds(0, 2048)], sos[q]
                    ).wait()

                for c in range(128):
                    row = plsc.load_gather(ins[q], [lanes, lanes128 * 0 + c])
                    outs[q][pl.ds(c * 16, 16)] = row
                off = pl.multiple_of(j * 2048, 8)
                pltpu.async_copy(outs[q], lin_hbm.at[pl.ds(off, 2048)], sos[q])

        return 0

    lax.fori_loop(0, (COLS_PER_W + 1) // 2, body, 0)

    pltpu.make_async_copy(out0, lin_hbm.at[pl.ds(0, 2048)], so0).wait()
    pltpu.make_async_copy(out1, lin_hbm.at[pl.ds(0, 2048)], so1).wait()

    @pl.when(wid == NFULL % NW)
    def _():
        pltpu.sync_copy(tail_hbm, out0.at[pl.ds(0, TAIL * DIM_EMB)])
        pltpu.sync_copy(
            out0.at[pl.ds(0, TAIL * DIM_EMB)],
            lin_hbm.at[pl.ds(NFULL * 128 * DIM_EMB, TAIL * DIM_EMB)],
        )


def _gather_body(table_hbm, ids_hbm, out_hbm, idx_v, rows_v, sem0, sem1):
    nc = 2
    wid = lax.axis_index("s") * nc + lax.axis_index("c")
    base = wid * RW
    pltpu.sync_copy(ids_hbm.at[pl.ds(base, RW)], idx_v)

    def fire(s, b, sem):
        soff = pl.multiple_of(s * SUPER, 8)
        for c in range(K):
            pltpu.async_copy(
                table_hbm.at[idx_v.at[pl.ds(soff + c * CHUNK, CHUNK)]],
                rows_v.at[b, pl.ds(c * CHUNK, CHUNK)],
                sem,
            )

    def drain(b, sem):
        for c in range(K):
            pltpu.make_async_copy(
                table_hbm.at[idx_v.at[pl.ds(c * CHUNK, CHUNK)]],
                rows_v.at[b, pl.ds(c * CHUNK, CHUNK)],
                sem,
            ).wait()

    def flush(s, b):
        ooff = pl.multiple_of(base + s * SUPER, 8)
        pltpu.sync_copy(rows_v.at[b], out_hbm.at[pl.ds(ooff, SUPER)])

    fire(0, 0, sem0)

    def pair(p, _):
        s0 = p * 2
        fire(s0 + 1, 1, sem1)
        drain(0, sem0)
        flush(s0, 0)

        @pl.when(s0 + 2 < NSUPER)
        def _():
            fire(s0 + 2, 0, sem0)

        drain(1, sem1)
        flush(s0 + 1, 1)
        return 0

    lax.fori_loop(0, NSUPER // 2, pair, 0)


@jax.jit
def _encode(emb_weight, flat_ids):
    mesh = plsc.VectorSubcoreMesh(core_axis_name="c", subcore_axis_name="s")
    detile = pl.kernel(
        _transpose_body,
        out_type=jax.ShapeDtypeStruct((N_UNIQUE * DIM_EMB,), jnp.float32),
        mesh=mesh,
        scratch_types=[
            pltpu.VMEM((DIM_EMB, 128), jnp.float32),
            pltpu.VMEM((DIM_EMB, 128), jnp.float32),
            pltpu.VMEM((128 * DIM_EMB,), jnp.float32),
            pltpu.VMEM((128 * DIM_EMB,), jnp.float32),
            pltpu.SemaphoreType.DMA,
            pltpu.SemaphoreType.DMA,
            pltpu.SemaphoreType.DMA,
            pltpu.SemaphoreType.DMA,
        ],
        compiler_params=pltpu.CompilerParams(needs_layout_passes=False),
    )
    tail = emb_weight[NFULL * 128 :].reshape(-1)
    lin = detile(emb_weight.T, tail)
    table = lin.reshape(N_UNIQUE, DIM_EMB)

    gather = pl.kernel(
        _gather_body,
        out_type=jax.ShapeDtypeStruct((R, DIM_EMB), jnp.float32),
        mesh=mesh,
        scratch_types=[
            pltpu.VMEM((RW,), jnp.int32),
            pltpu.VMEM((2, SUPER, DIM_EMB), jnp.float32),
            pltpu.SemaphoreType.DMA,
            pltpu.SemaphoreType.DMA,
        ],
        compiler_params=pltpu.CompilerParams(use_tc_tiling_on_sc=False),
    )
    return gather(table, flat_ids)


def kernel(state_ids, emb_weight):
    flat_ids = state_ids.reshape(-1)
    out = _encode(emb_weight, flat_ids)
    return out.reshape(BATCH, N_FIELDS * DIM_EMB)


# speedup vs baseline: 2.0357x; 1.4641x over previous
import jax
import jax.numpy as jnp
from jax import lax
from jax.experimental import pallas as pl
from jax.experimental.pallas import tpu as pltpu
from jax.experimental.pallas import tpu_sc as plsc

N_UNIQUE = 1000000
DIM_EMB = 16
BATCH = 16384
N_FIELDS = 26

R = BATCH * N_FIELDS
NW = 32
RW = R // NW
CHUNK = 128
K = 13
SUPER = K * CHUNK
NSUPER = RW // SUPER

NFULL = N_UNIQUE // 128
TAIL = N_UNIQUE - NFULL * 128
BLOCKC = 4
BCOLS = BLOCKC * 128
NBLK = NFULL // BLOCKC
BLK_PER_W = (NBLK + NW - 1) // NW


def _transpose_body(
    wt_hbm, tail_hbm, lin_hbm, in0, in1, out0, out1, si0, si1, so0, so1
):
    nc = 2
    wid = lax.axis_index("s") * nc + lax.axis_index("c")
    lanes = lax.iota(jnp.int32, 16)
    ins = (in0, in1)
    outs = (out0, out1)
    sis = (si0, si1)
    sos = (so0, so1)

    def fire_in(i, q):
        off = pl.multiple_of((i * NW + wid) * BCOLS, 128)
        pltpu.async_copy(wt_hbm.at[:, pl.ds(off, BCOLS)], ins[q], sis[q])

    fire_in(0, 0)

    def body(p, _):
        for q in (0, 1):
            i = p * 2 + q
            j = i * NW + wid

            @pl.when(j < NBLK)
            def _():
                @pl.when(j + NW < NBLK)
                def _():
                    fire_in(i + 1, 1 - q)

                pltpu.make_async_copy(
                    wt_hbm.at[:, pl.ds(0, BCOLS)], ins[q], sis[q]
                ).wait()

                @pl.when(j >= 2 * NW)
                def _():
                    pltpu.make_async_copy(
                        outs[q], lin_hbm.at[pl.ds(0, BCOLS * DIM_EMB)], sos[q]
                    ).wait()

                @plsc.parallel_loop(0, BCOLS, unroll=8)
                def _(c):
                    row = plsc.load_gather(ins[q], [lanes, lanes * 0 + c])
                    outs[q][pl.ds(c * DIM_EMB, DIM_EMB)] = row

                off = pl.multiple_of(j * BCOLS * DIM_EMB, 8)
                pltpu.async_copy(
                    outs[q], lin_hbm.at[pl.ds(off, BCOLS * DIM_EMB)], sos[q]
                )

        return 0

    lax.fori_loop(0, (BLK_PER_W + 1) // 2, body, 0)

    pltpu.make_async_copy(out0, lin_hbm.at[pl.ds(0, BCOLS * DIM_EMB)], so0).wait()
    pltpu.make_async_copy(out1, lin_hbm.at[pl.ds(0, BCOLS * DIM_EMB)], so1).wait()

    @pl.when(wid == NFULL % NW)
    def _():
        pltpu.sync_copy(tail_hbm, out0.at[pl.ds(0, TAIL * DIM_EMB)])
        pltpu.sync_copy(
            out0.at[pl.ds(0, TAIL * DIM_EMB)],
            lin_hbm.at[pl.ds(NFULL * 128 * DIM_EMB, TAIL * DIM_EMB)],
        )


def _gather_body(table_hbm, ids_hbm, out_hbm, idx_v, rows_v, sem0, sem1):
    nc = 2
    wid = lax.axis_index("s") * nc + lax.axis_index("c")
    base = wid * RW
    pltpu.sync_copy(ids_hbm.at[pl.ds(base, RW)], idx_v)

    def fire(s, b, sem):
        soff = pl.multiple_of(s * SUPER, 8)
        for c in range(K):
            pltpu.async_copy(
                table_hbm.at[idx_v.at[pl.ds(soff + c * CHUNK, CHUNK)]],
                rows_v.at[b, pl.ds(c * CHUNK, CHUNK)],
                sem,
            )

    def drain(b, sem):
        for c in range(K):
            pltpu.make_async_copy(
                table_hbm.at[idx_v.at[pl.ds(c * CHUNK, CHUNK)]],
                rows_v.at[b, pl.ds(c * CHUNK, CHUNK)],
                sem,
            ).wait()

    def flush(s, b):
        ooff = pl.multiple_of(base + s * SUPER, 8)
        pltpu.sync_copy(rows_v.at[b], out_hbm.at[pl.ds(ooff, SUPER)])

    fire(0, 0, sem0)

    def pair(p, _):
        s0 = p * 2
        fire(s0 + 1, 1, sem1)
        drain(0, sem0)
        flush(s0, 0)

        @pl.when(s0 + 2 < NSUPER)
        def _():
            fire(s0 + 2, 0, sem0)

        drain(1, sem1)
        flush(s0 + 1, 1)
        return 0

    lax.fori_loop(0, NSUPER // 2, pair, 0)


@jax.jit
def _encode(emb_weight, flat_ids):
    mesh = plsc.VectorSubcoreMesh(core_axis_name="c", subcore_axis_name="s")
    detile = pl.kernel(
        _transpose_body,
        out_type=jax.ShapeDtypeStruct((N_UNIQUE * DIM_EMB,), jnp.float32),
        mesh=mesh,
        scratch_types=[
            pltpu.VMEM((DIM_EMB, BCOLS), jnp.float32),
            pltpu.VMEM((DIM_EMB, BCOLS), jnp.float32),
            pltpu.VMEM((BCOLS * DIM_EMB,), jnp.float32),
            pltpu.VMEM((BCOLS * DIM_EMB,), jnp.float32),
            pltpu.SemaphoreType.DMA,
            pltpu.SemaphoreType.DMA,
            pltpu.SemaphoreType.DMA,
            pltpu.SemaphoreType.DMA,
        ],
        compiler_params=pltpu.CompilerParams(needs_layout_passes=False),
    )
    tail = emb_weight[NFULL * 128 :].reshape(-1)
    lin = detile(emb_weight.T, tail)
    table = lin.reshape(N_UNIQUE, DIM_EMB)

    gather = pl.kernel(
        _gather_body,
        out_type=jax.ShapeDtypeStruct((R, DIM_EMB), jnp.float32),
        mesh=mesh,
        scratch_types=[
            pltpu.VMEM((RW,), jnp.int32),
            pltpu.VMEM((2, SUPER, DIM_EMB), jnp.float32),
            pltpu.SemaphoreType.DMA,
            pltpu.SemaphoreType.DMA,
        ],
        compiler_params=pltpu.CompilerParams(use_tc_tiling_on_sc=False),
    )
    return gather(table, flat_ids)


def kernel(state_ids, emb_weight):
    flat_ids = state_ids.reshape(-1)
    out = _encode(emb_weight, flat_ids)
    return out.reshape(BATCH, N_FIELDS * DIM_EMB)


# speedup vs baseline: 2.0367x; 1.0005x over previous
import jax
import jax.numpy as jnp
from jax import lax
from jax.experimental import pallas as pl
from jax.experimental.pallas import tpu as pltpu
from jax.experimental.pallas import tpu_sc as plsc

N_UNIQUE = 1000000
DIM_EMB = 16
BATCH = 16384
N_FIELDS = 26

R = BATCH * N_FIELDS
NW = 32
RW = R // NW
CHUNK = 128
K = 13
SUPER = K * CHUNK
NSUPER = RW // SUPER

NFULL = N_UNIQUE // 128
TAIL = N_UNIQUE - NFULL * 128
BLOCKC = 4
BCOLS = BLOCKC * 128
NBLK = NFULL // BLOCKC
BLK_PER_W = (NBLK + NW - 1) // NW


def _transpose_body(
    wt_hbm, tail_hbm, lin_hbm, in0, in1, out0, out1, si0, si1, so0, so1
):
    nc = 2
    wid = lax.axis_index("s") * nc + lax.axis_index("c")
    lanes = lax.iota(jnp.int32, 16)
    ins = (in0, in1)
    outs = (out0, out1)
    sis = (si0, si1)
    sos = (so0, so1)

    def fire_in(i, q):
        off = pl.multiple_of((i * NW + wid) * BCOLS, 128)
        pltpu.async_copy(
            wt_hbm.at[:, pl.ds(off, BCOLS)], ins[q].at[:, pl.ds(0, BCOLS)], sis[q]
        )

    fire_in(0, 0)

    def body(p, _):
        for q in (0, 1):
            i = p * 2 + q
            j = i * NW + wid

            @pl.when(j < NBLK)
            def _():
                @pl.when(j + NW < NBLK)
                def _():
                    fire_in(i + 1, 1 - q)

                pltpu.make_async_copy(
                    wt_hbm.at[:, pl.ds(0, BCOLS)],
                    ins[q].at[:, pl.ds(0, BCOLS)],
                    sis[q],
                ).wait()

                @pl.when(j >= 2 * NW)
                def _():
                    pltpu.make_async_copy(
                        outs[q], lin_hbm.at[pl.ds(0, BCOLS * DIM_EMB)], sos[q]
                    ).wait()

                @plsc.parallel_loop(0, BCOLS, unroll=8)
                def _(c):
                    row = plsc.load_gather(ins[q], [lanes, lanes * 0 + c])
                    outs[q][pl.ds(c * DIM_EMB, DIM_EMB)] = row

                off = pl.multiple_of(j * BCOLS * DIM_EMB, 8)
                pltpu.async_copy(
                    outs[q], lin_hbm.at[pl.ds(off, BCOLS * DIM_EMB)], sos[q]
                )

        return 0

    lax.fori_loop(0, (BLK_PER_W + 1) // 2, body, 0)

    pltpu.make_async_copy(out0, lin_hbm.at[pl.ds(0, BCOLS * DIM_EMB)], so0).wait()
    pltpu.make_async_copy(out1, lin_hbm.at[pl.ds(0, BCOLS * DIM_EMB)], so1).wait()

    @pl.when(wid == NFULL % NW)
    def _():
        pltpu.sync_copy(tail_hbm, out0.at[pl.ds(0, TAIL * DIM_EMB)])
        pltpu.sync_copy(
            out0.at[pl.ds(0, TAIL * DIM_EMB)],
            lin_hbm.at[pl.ds(NFULL * 128 * DIM_EMB, TAIL * DIM_EMB)],
        )


def _gather_body(table_hbm, ids_hbm, out_hbm, idx_v, rows_v, sem0, sem1):
    nc = 2
    wid = lax.axis_index("s") * nc + lax.axis_index("c")
    base = wid * RW
    pltpu.sync_copy(ids_hbm.at[pl.ds(base, RW)], idx_v)

    def fire(s, b, sem):
        soff = pl.multiple_of(s * SUPER, 8)
        for c in range(K):
            pltpu.async_copy(
                table_hbm.at[idx_v.at[pl.ds(soff + c * CHUNK, CHUNK)]],
                rows_v.at[b, pl.ds(c * CHUNK, CHUNK)],
                sem,
            )

    def drain(b, sem):
        for c in range(K):
            pltpu.make_async_copy(
                table_hbm.at[idx_v.at[pl.ds(c * CHUNK, CHUNK)]],
                rows_v.at[b, pl.ds(c * CHUNK, CHUNK)],
                sem,
            ).wait()

    def flush(s, b):
        ooff = pl.multiple_of(base + s * SUPER, 8)
        pltpu.sync_copy(rows_v.at[b], out_hbm.at[pl.ds(ooff, SUPER)])

    fire(0, 0, sem0)

    def pair(p, _):
        s0 = p * 2
        fire(s0 + 1, 1, sem1)
        drain(0, sem0)
        flush(s0, 0)

        @pl.when(s0 + 2 < NSUPER)
        def _():
            fire(s0 + 2, 0, sem0)

        drain(1, sem1)
        flush(s0 + 1, 1)
        return 0

    lax.fori_loop(0, NSUPER // 2, pair, 0)


@jax.jit
def _encode(emb_weight, flat_ids):
    mesh = plsc.VectorSubcoreMesh(core_axis_name="c", subcore_axis_name="s")
    detile = pl.kernel(
        _transpose_body,
        out_type=jax.ShapeDtypeStruct((N_UNIQUE * DIM_EMB,), jnp.float32),
        mesh=mesh,
        scratch_types=[
            pltpu.VMEM((DIM_EMB, BCOLS + 1), jnp.float32),
            pltpu.VMEM((DIM_EMB, BCOLS + 1), jnp.float32),
            pltpu.VMEM((BCOLS * DIM_EMB,), jnp.float32),
            pltpu.VMEM((BCOLS * DIM_EMB,), jnp.float32),
            pltpu.SemaphoreType.DMA,
            pltpu.SemaphoreType.DMA,
            pltpu.SemaphoreType.DMA,
            pltpu.SemaphoreType.DMA,
        ],
        compiler_params=pltpu.CompilerParams(needs_layout_passes=False),
    )
    tail = emb_weight[NFULL * 128 :].reshape(-1)
    lin = detile(emb_weight.T, tail)
    table = lin.reshape(N_UNIQUE, DIM_EMB)

    gather = pl.kernel(
        _gather_body,
        out_type=jax.ShapeDtypeStruct((R, DIM_EMB), jnp.float32),
        mesh=mesh,
        scratch_types=[
            pltpu.VMEM((RW,), jnp.int32),
            pltpu.VMEM((2, SUPER, DIM_EMB), jnp.float32),
            pltpu.SemaphoreType.DMA,
            pltpu.SemaphoreType.DMA,
        ],
        compiler_params=pltpu.CompilerParams(use_tc_tiling_on_sc=False),
    )
    return gather(table, flat_ids)


def kernel(state_ids, emb_weight):
    flat_ids = state_ids.reshape(-1)
    out = _encode(emb_weight, flat_ids)
    return out.reshape(BATCH, N_FIELDS * DIM_EMB)


# speedup vs baseline: 2.2090x; 1.0846x over previous
import jax
import jax.numpy as jnp
from jax import lax
from jax.experimental import pallas as pl
from jax.experimental.pallas import tpu as pltpu
from jax.experimental.pallas import tpu_sc as plsc

N_UNIQUE = 1000000
DIM_EMB = 16
BATCH = 16384
N_FIELDS = 26

R = BATCH * N_FIELDS
NW = 32
RW = R // NW
CHUNK = 128
K = 13
SUPER = K * CHUNK
NSUPER = RW // SUPER

NFULL = N_UNIQUE // 128
TAIL = N_UNIQUE - NFULL * 128
BLOCKC = 4
BCOLS = BLOCKC * 128
NBLK = NFULL // BLOCKC
BLK_PER_W = (NBLK + NW - 1) // NW


def _transpose_body(
    wt_hbm, tail_hbm, lin_hbm, in0, in1, out0, out1, si0, si1, so0, so1
):
    nc = 2
    wid = lax.axis_index("s") * nc + lax.axis_index("c")
    lanes = lax.iota(jnp.int32, 16)
    ins = (in0, in1)
    outs = (out0, out1)
    sis = (si0, si1)
    sos = (so0, so1)

    def fire_in(i, q):
        off = pl.multiple_of((i * NW + wid) * BCOLS, 128)
        pltpu.async_copy(
            wt_hbm.at[:, pl.ds(off, BCOLS)], ins[q].at[:, pl.ds(0, BCOLS)], sis[q]
        )

    fire_in(0, 0)

    def body(p, _):
        for q in (0, 1):
            i = p * 2 + q
            j = i * NW + wid

            @pl.when(j < NBLK)
            def _():
                @pl.when(j + NW < NBLK)
                def _():
                    fire_in(i + 1, 1 - q)

                pltpu.make_async_copy(
                    wt_hbm.at[:, pl.ds(0, BCOLS)],
                    ins[q].at[:, pl.ds(0, BCOLS)],
                    sis[q],
                ).wait()

                @pl.when(j >= 2 * NW)
                def _():
                    pltpu.make_async_copy(
                        outs[q], lin_hbm.at[pl.ds(0, BCOLS * DIM_EMB)], sos[q]
                    ).wait()

                @plsc.parallel_loop(0, BCOLS, unroll=16, carry=lanes * 0)
                def _(c, colv):
                    row = plsc.load_gather(ins[q], [lanes, colv])
                    outs[q][pl.ds(c * DIM_EMB, DIM_EMB)] = row
                    return colv + 1

                off = pl.multiple_of(j * BCOLS * DIM_EMB, 8)
                pltpu.async_copy(
                    outs[q], lin_hbm.at[pl.ds(off, BCOLS * DIM_EMB)], sos[q]
                )

        return 0

    lax.fori_loop(0, (BLK_PER_W + 1) // 2, body, 0)

    pltpu.make_async_copy(out0, lin_hbm.at[pl.ds(0, BCOLS * DIM_EMB)], so0).wait()
    pltpu.make_async_copy(out1, lin_hbm.at[pl.ds(0, BCOLS * DIM_EMB)], so1).wait()

    @pl.when(wid == NFULL % NW)
    def _():
        pltpu.sync_copy(tail_hbm, out0.at[pl.ds(0, TAIL * DIM_EMB)])
        pltpu.sync_copy(
            out0.at[pl.ds(0, TAIL * DIM_EMB)],
            lin_hbm.at[pl.ds(NFULL * 128 * DIM_EMB, TAIL * DIM_EMB)],
        )


def _gather_body(table_hbm, ids_hbm, out_hbm, idx_v, rows_v, sem0, sem1):
    nc = 2
    wid = lax.axis_index("s") * nc + lax.axis_index("c")
    base = wid * RW
    pltpu.sync_copy(ids_hbm.at[pl.ds(base, RW)], idx_v)

    def fire(s, b, sem):
        soff = pl.multiple_of(s * SUPER, 8)
        for c in range(K):
            pltpu.async_copy(
                table_hbm.at[idx_v.at[pl.ds(soff + c * CHUNK, CHUNK)]],
                rows_v.at[b, pl.ds(c * CHUNK, CHUNK)],
                sem,
            )

    def drain(b, sem):
        for c in range(K):
            pltpu.make_async_copy(
                table_hbm.at[idx_v.at[pl.ds(c * CHUNK, CHUNK)]],
                rows_v.at[b, pl.ds(c * CHUNK, CHUNK)],
                sem,
            ).wait()

    def flush(s, b):
        ooff = pl.multiple_of(base + s * SUPER, 8)
        pltpu.sync_copy(rows_v.at[b], out_hbm.at[pl.ds(ooff, SUPER)])

    fire(0, 0, sem0)

    def pair(p, _):
        s0 = p * 2
        fire(s0 + 1, 1, sem1)
        drain(0, sem0)
        flush(s0, 0)

        @pl.when(s0 + 2 < NSUPER)
        def _():
            fire(s0 + 2, 0, sem0)

        drain(1, sem1)
        flush(s0 + 1, 1)
        return 0

    lax.fori_loop(0, NSUPER // 2, pair, 0)


@jax.jit
def _encode(emb_weight, flat_ids):
    mesh = plsc.VectorSubcoreMesh(core_axis_name="c", subcore_axis_name="s")
    detile = pl.kernel(
        _transpose_body,
        out_type=jax.ShapeDtypeStruct((N_UNIQUE * DIM_EMB,), jnp.float32),
        mesh=mesh,
        scratch_types=[
            pltpu.VMEM((DIM_EMB, BCOLS + 1), jnp.float32),
            pltpu.VMEM((DIM_EMB, BCOLS + 1), jnp.float32),
            pltpu.VMEM((BCOLS * DIM_EMB,), jnp.float32),
            pltpu.VMEM((BCOLS * DIM_EMB,), jnp.float32),
            pltpu.SemaphoreType.DMA,
            pltpu.SemaphoreType.DMA,
            pltpu.SemaphoreType.DMA,
            pltpu.SemaphoreType.DMA,
        ],
        compiler_params=pltpu.CompilerParams(needs_layout_passes=False),
    )
    tail = emb_weight[NFULL * 128 :].reshape(-1)
    lin = detile(emb_weight.T, tail)
    table = lin.reshape(N_UNIQUE, DIM_EMB)

    gather = pl.kernel(
        _gather_body,
        out_type=jax.ShapeDtypeStruct((R, DIM_EMB), jnp.float32),
        mesh=mesh,
        scratch_types=[
            pltpu.VMEM((RW,), jnp.int32),
            pltpu.VMEM((2, SUPER, DIM_EMB), jnp.float32),
            pltpu.SemaphoreType.DMA,
            pltpu.SemaphoreType.DMA,
        ],
        compiler_params=pltpu.CompilerParams(use_tc_tiling_on_sc=False),
    )
    return gather(table, flat_ids)


def kernel(state_ids, emb_weight):
    flat_ids = state_ids.reshape(-1)
    out = _encode(emb_weight, flat_ids)
    return out.reshape(BATCH, N_FIELDS * DIM_EMB)
